# Initial kernel scaffold; baseline (speedup 1.0000x reference)
import jax
import jax.numpy as jnp
from jax.experimental import pallas as pl


def kernel(x, edge_index, W1, b1, W2, b2, W3, b3):
    raise NotImplementedError("write your pallas kernel here")


# speedup vs baseline: 8.9812x; 8.9812x over previous
import functools

import jax
import jax.numpy as jnp
from jax import lax
from jax.experimental import pallas as pl
from jax.experimental.pallas import tpu as pltpu
from jax.experimental.pallas import tpu_sc as plsc

NC = 2
NS = 16
CHUNK = 128
RB = 1000


def _mesh():
    return plsc.VectorSubcoreMesh(
        core_axis_name="c", subcore_axis_name="s", num_cores=NC, num_subcores=NS
    )


def _copy_rows(s, src_ref, dst_ref, src_base, dst_base, N):
    rps = (N // NS) // 8 * 8
    tail = N - NS * rps
    pltpu.sync_copy(src_ref.at[pl.ds(src_base + s * rps, rps)],
                    dst_ref.at[pl.ds(dst_base + s * rps, rps)])
    if tail:
        @pl.when(s == NS - 1)
        def _():
            pltpu.sync_copy(src_ref.at[pl.ds(src_base + NS * rps, tail)],
                            dst_ref.at[pl.ds(dst_base + NS * rps, tail)])


def _make_deg(N, CPW):
    @functools.partial(
        pl.kernel,
        out_type=jax.ShapeDtypeStruct((2 * N, 16), jnp.float32),
        mesh=_mesh(),
        scratch_types=[
            pltpu.VMEM((CHUNK,), jnp.int32),
            pltpu.VMEM((CHUNK, 16), jnp.float32),
            pltpu.VMEM_SHARED((N + 16, 16), jnp.float32),
        ],
    )
    def k(dstp, ones_hbm, zeros_hbm, out, didx, ones_v, accum):
        c = lax.axis_index("c")
        s = lax.axis_index("s")
        pltpu.sync_copy(ones_hbm, ones_v)
        _copy_rows(s, zeros_hbm, accum, 0, 0, N)
        plsc.subcore_barrier()

        def body(j, carry):
            base = ((c * NS + s) * CPW + j) * CHUNK
            pltpu.sync_copy(dstp.at[pl.ds(base, CHUNK)], didx)
            pltpu.sync_copy(ones_v, accum.at[didx], add=True)
            return carry

        lax.fori_loop(0, CPW, body, 0)
        plsc.subcore_barrier()
        _copy_rows(s, accum, out, 0, c * N, N)

    return k


def _make_agg(N, F, CPW):
    @functools.partial(
        pl.kernel,
        out_type=jax.ShapeDtypeStruct((2 * N, F), jnp.float32),
        mesh=_mesh(),
        scratch_types=[
            pltpu.VMEM((CHUNK,), jnp.int32),
            pltpu.VMEM((CHUNK,), jnp.int32),
            pltpu.VMEM((CHUNK, F), jnp.float32),
            pltpu.VMEM_SHARED((N + 16, F), jnp.float32),
        ],
    )
    def k(hp, srcp, dstp, zeros_hbm, out, sidx, didx, rows, accum):
        c = lax.axis_index("c")
        s = lax.axis_index("s")

        @pl.when(c == 0)
        def _():
            _copy_rows(s, hp, accum, 0, 0, N)

        @pl.when(c == 1)
        def _():
            _copy_rows(s, zeros_hbm, accum, 0, 0, N)

        plsc.subcore_barrier()

        def body(j, carry):
            base = ((c * NS + s) * CPW + j) * CHUNK
            pltpu.sync_copy(srcp.at[pl.ds(base, CHUNK)], sidx)
            pltpu.sync_copy(dstp.at[pl.ds(base, CHUNK)], didx)
            pltpu.sync_copy(hp.at[sidx], rows)
            pltpu.sync_copy(rows, accum.at[didx], add=True)
            return carry

        lax.fori_loop(0, CPW, body, 0)
        plsc.subcore_barrier()
        _copy_rows(s, accum, out, 0, c * N, N)

    return k


def _tc_first(x, w1):
    N, F = x.shape
    H = w1.shape[1]

    def body(x_ref, w_ref, out_ref):
        out_ref[...] = jnp.dot(x_ref[...], w_ref[...],
                               preferred_element_type=jnp.float32)

    return pl.pallas_call(
        body,
        grid=(N // RB,),
        in_specs=[
            pl.BlockSpec((RB, F), lambda i: (i, 0)),
            pl.BlockSpec((F, H), lambda i: (0, 0)),
        ],
        out_specs=pl.BlockSpec((RB, H), lambda i: (i, 0)),
        out_shape=jax.ShapeDtypeStruct((N, H), jnp.float32),
    )(x, w1)


def _tc_dinv(degp, xw1):
    _, N, _ = degp.shape
    H = xw1.shape[1]

    def body(degp_ref, xw1_ref, dinv_ref, h1p_ref):
        d = degp_ref[0, :, 0:1] + degp_ref[1, :, 0:1] + 1.0
        dv = jnp.broadcast_to(lax.rsqrt(d), (RB, H))
        dinv_ref[...] = dv
        h1p_ref[...] = xw1_ref[...] * dv

    return pl.pallas_call(
        body,
        grid=(N // RB,),
        in_specs=[
            pl.BlockSpec((2, RB, 16), lambda i: (0, i, 0)),
            pl.BlockSpec((RB, H), lambda i: (i, 0)),
        ],
        out_specs=[
            pl.BlockSpec((RB, H), lambda i: (i, 0)),
            pl.BlockSpec((RB, H), lambda i: (i, 0)),
        ],
        out_shape=[
            jax.ShapeDtypeStruct((N, H), jnp.float32),
            jax.ShapeDtypeStruct((N, H), jnp.float32),
        ],
    )(degp, xw1)


def _tc_mid(p, dinv, w, b):
    _, N, H = p.shape
    Ho = w.shape[1]

    def body(p_ref, d_ref, w_ref, b_ref, out_ref):
        d = d_ref[...]
        t = jnp.maximum((p_ref[0] + p_ref[1]) * d + b_ref[...], 0.0)
        out_ref[...] = jnp.dot(t, w_ref[...],
                               preferred_element_type=jnp.float32) * d

    return pl.pallas_call(
        body,
        grid=(N // RB,),
        in_specs=[
            pl.BlockSpec((2, RB, H), lambda i: (0, i, 0)),
            pl.BlockSpec((RB, H), lambda i: (i, 0)),
            pl.BlockSpec((H, Ho), lambda i: (0, 0)),
            pl.BlockSpec((1, H), lambda i: (0, 0)),
        ],
        out_specs=pl.BlockSpec((RB, Ho), lambda i: (i, 0)),
        out_shape=jax.ShapeDtypeStruct((N, Ho), jnp.float32),
    )(p, dinv, w, b)


def _tc_final(p, dinv, b):
    _, N, H = p.shape

    def body(p_ref, d_ref, b_ref, out_ref):
        out_ref[...] = (p_ref[0] + p_ref[1]) * d_ref[...] + b_ref[...]

    return pl.pallas_call(
        body,
        grid=(N // RB,),
        in_specs=[
            pl.BlockSpec((2, RB, H), lambda i: (0, i, 0)),
            pl.BlockSpec((RB, H), lambda i: (i, 0)),
            pl.BlockSpec((1, H), lambda i: (0, 0)),
        ],
        out_specs=pl.BlockSpec((RB, H), lambda i: (i, 0)),
        out_shape=jax.ShapeDtypeStruct((N, H), jnp.float32),
    )(p, dinv, b)


@jax.jit
def kernel(x, edge_index, W1, b1, W2, b2, W3, b3):
    N, F = x.shape
    E = edge_index.shape[1]
    H = W1.shape[1]
    C = W3.shape[1]

    src = edge_index[0]
    dst = edge_index[1]

    CPW = -(-E // (NC * NS * CHUNK))
    epad = CPW * NC * NS * CHUNK
    src_p = jnp.concatenate([src, jnp.zeros((epad - E,), jnp.int32)])
    dst_p = jnp.concatenate([dst, jnp.full((epad - E,), N, jnp.int32)])
    ones16 = jnp.ones((CHUNK, 16), jnp.float32)
    zerosNH = jnp.zeros((N, H), jnp.float32)
    W3p = jnp.concatenate([W3, jnp.zeros((H, H - C), jnp.float32)], axis=1)
    b3p = jnp.concatenate([b3, jnp.zeros((H - C,), jnp.float32)])

    deg = _make_deg(N, CPW)
    agg = _make_agg(N, H, CPW)

    xw1 = _tc_first(x, W1)
    degp = deg(dst_p, ones16, zerosNH[:, :16])
    dinv, h1p = _tc_dinv(degp.reshape(2, N, 16), xw1)

    acc1 = agg(h1p, src_p, dst_p, zerosNH)
    h2p = _tc_mid(acc1.reshape(2, N, H), dinv, W2, b1[None, :])

    acc2 = agg(h2p, src_p, dst_p, zerosNH)
    h3p = _tc_mid(acc2.reshape(2, N, H), dinv, W3p, b2[None, :])

    acc3 = agg(h3p, src_p, dst_p, zerosNH)
    out = _tc_final(acc3.reshape(2, N, H), dinv, b3p[None, :])
    return out[:, :C]
